# baseline (speedup 1.0000x reference)
import jax
import jax.numpy as jnp
from jax.experimental import pallas as pl
from jax.experimental.pallas import tpu as pltpu

N, D, H = 50000, 256, 256
BLOCK_ROWS = 10000
NUM_BLOCKS = N // BLOCK_ROWS


def _fused_kernel(x_ref, vx_ref, w1_ref, b1_ref, gamma_ref, beta_ref,
                  h_ref, vxnew_ref, acc_ref):
    i = pl.program_id(0)
    vx = vx_ref[...]
    hb = x_ref[...] + vx
    h_ref[...] = hb
    bsum = jnp.sum(hb, axis=0, keepdims=True)

    @pl.when(i == 0)
    def _init():
        acc_ref[...] = bsum

    @pl.when(i > 0)
    def _acc():
        acc_ref[...] = acc_ref[...] + bsum

    @pl.when(i == NUM_BLOCKS - 1)
    def _epilogue():
        vx_temp = acc_ref[...] + vx
        z = jnp.dot(vx_temp, w1_ref[...],
                    preferred_element_type=jnp.float32) + b1_ref[...]
        mu = jnp.mean(z, axis=-1, keepdims=True)
        var = jnp.mean((z - mu) * (z - mu), axis=-1, keepdims=True)
        zn = gamma_ref[...] * (z - mu) * jax.lax.rsqrt(var + 1e-5) + beta_ref[...]
        vxnew_ref[...] = jnp.maximum(zn, 0.0)


@jax.jit
def kernel(x, vn_emb, W1, b1, gamma, beta):
    b1r = b1.reshape(1, H)
    gr = gamma.reshape(1, H)
    br = beta.reshape(1, H)
    h, vx_new = pl.pallas_call(
        _fused_kernel,
        grid=(NUM_BLOCKS,),
        in_specs=[
            pl.BlockSpec((BLOCK_ROWS, D), lambda i: (i, 0)),
            pl.BlockSpec((1, D), lambda i: (0, 0)),
            pl.BlockSpec((D, H), lambda i: (0, 0)),
            pl.BlockSpec((1, H), lambda i: (0, 0)),
            pl.BlockSpec((1, H), lambda i: (0, 0)),
            pl.BlockSpec((1, H), lambda i: (0, 0)),
        ],
        out_specs=[
            pl.BlockSpec((BLOCK_ROWS, D), lambda i: (i, 0)),
            pl.BlockSpec((1, H), lambda i: (0, 0)),
        ],
        out_shape=[
            jax.ShapeDtypeStruct((N, D), jnp.float32),
            jax.ShapeDtypeStruct((1, H), jnp.float32),
        ],
        scratch_shapes=[pltpu.VMEM((1, D), jnp.float32)],
    )(x, vn_emb, W1, b1r, gr, br)
    return (h, vx_new)
